# speedup vs baseline: 1.4899x; 1.4899x over previous
import jax
import jax.numpy as jnp
from jax import lax
from jax.experimental import pallas as pl
from jax.experimental.pallas import tpu as pltpu

_V = 100000
_D = 16
_A = 1024
_B = 1024

_VT = 2048
_T = (_V + _VT - 1) // _VT
_V_PAD = _T * _VT


def _body(idx_ref, adap_ref, embT_ref, emb_ref, out_ref, a_scr, m_scr):
    t = pl.program_id(0)

    @pl.when(t == 0)
    def _init():
        eidx = jnp.maximum(idx_ref[...] - _V, 0)
        aiota = lax.broadcasted_iota(jnp.int32, (_B, _A), 1)
        onehot = (eidx == aiota).astype(jnp.float32)
        a_scr[...] = lax.dot_general(
            onehot, adap_ref[...], (((1,), (0,)), ((), ())),
            precision=lax.Precision.HIGHEST,
            preferred_element_type=jnp.float32)
        m_scr[...] = jnp.full((_B, 1), -jnp.inf, jnp.float32)
        out_ref[...] = jnp.zeros((_B, _D), jnp.float32)

    a = a_scr[...]
    rel = lax.dot_general(a, embT_ref[...], (((1,), (0,)), ((), ())),
                          preferred_element_type=jnp.float32)
    col = t * _VT + lax.broadcasted_iota(jnp.int32, (1, _VT), 1)
    rel = jnp.where(col < _V, rel, -jnp.inf)
    tile_max = jnp.max(rel, axis=1, keepdims=True)
    m_old = m_scr[...]
    m_new = jnp.maximum(m_old, tile_max)
    mask = (rel == m_new).astype(jnp.float32)
    contrib = lax.dot_general(mask, emb_ref[...], (((1,), (0,)), ((), ())),
                              preferred_element_type=jnp.float32)
    out_ref[...] = jnp.where(tile_max > m_old, contrib, out_ref[...] + contrib)
    m_scr[...] = m_new


def kernel(items_indices, constant_table, adaptive_table):
    idx2 = items_indices.reshape(_B, 1)
    tpad = jnp.pad(constant_table, ((0, _V_PAD - _V), (0, 0)))
    tT = tpad.T
    return pl.pallas_call(
        _body,
        grid=(_T,),
        in_specs=[
            pl.BlockSpec((_B, 1), lambda t: (0, 0)),
            pl.BlockSpec((_A, _D), lambda t: (0, 0)),
            pl.BlockSpec((_D, _VT), lambda t: (0, t)),
            pl.BlockSpec((_VT, _D), lambda t: (t, 0)),
        ],
        out_specs=pl.BlockSpec((_B, _D), lambda t: (0, 0)),
        out_shape=jax.ShapeDtypeStruct((_B, _D), jnp.float32),
        scratch_shapes=[pltpu.VMEM((_B, _D), jnp.float32),
                        pltpu.VMEM((_B, 1), jnp.float32)],
    )(idx2, adaptive_table, tT, tpad)


# speedup vs baseline: 3.2513x; 2.1822x over previous
import jax
import jax.numpy as jnp
from jax import lax
from jax.experimental import pallas as pl
from jax.experimental.pallas import tpu as pltpu

_V = 100000
_D = 16
_A = 1024
_B = 1024

_VT = 8192
_T = (_V + _VT - 1) // _VT
_V_PAD = _T * _VT
_CH = 32
_U = _A

_HI = lax.Precision.HIGHEST
_STD = (((1,), (0,)), ((), ()))
_RT = (((1,), (1,)), ((), ()))


def _body(idx_ref, adap_ref, embT_ref, emb_ref, out_ref,
          au_scr, m_scr, acc_scr, rb_scr, n_scr):
    t = pl.program_id(0)

    @pl.when(t == 0)
    def _init():
        eidx = jnp.maximum(idx_ref[...] - _V, 0)
        aiota = lax.broadcasted_iota(jnp.int32, (_B, _A), 1)
        oh_ba = (eidx == aiota).astype(jnp.float32)
        ones_r = jnp.ones((1, _B), jnp.float32)
        counts = lax.dot_general(ones_r, oh_ba, _STD, precision=_HI)
        pres = (counts > 0.0).astype(jnp.float32)
        i_col = lax.broadcasted_iota(jnp.int32, (_A, _A), 0)
        j_row = lax.broadcasted_iota(jnp.int32, (_A, _A), 1)
        gt = (i_col < j_row).astype(jnp.float32)
        lt = (j_row < i_col).astype(jnp.float32)
        rank_r = lax.dot_general(pres, gt, _STD, precision=_HI)
        rank_c = lax.dot_general(lt, pres, _RT, precision=_HI)
        riota = lax.broadcasted_iota(jnp.int32, (_U, _A), 0)
        sel = ((riota == rank_r.astype(jnp.int32)) &
               (pres > 0.0)).astype(jnp.float32)
        au_scr[...] = lax.dot_general(sel, adap_ref[...], _STD,
                                      precision=_HI)
        rb_scr[...] = lax.dot_general(oh_ba, rank_c, _STD,
                                      precision=_HI)
        n_scr[0] = jnp.sum(pres).astype(jnp.int32)
        m_scr[...] = jnp.full((_U, 1), -jnp.inf, jnp.float32)
        acc_scr[...] = jnp.zeros((_U, _D), jnp.float32)

    embT = embT_ref[...]
    emb = emb_ref[...]
    col = t * _VT + lax.broadcasted_iota(jnp.int32, (1, _VT), 1)
    valid = col < _V
    n_ch = (n_scr[0] + _CH - 1) // _CH

    def _chunk(c, carry):
        rows = pl.ds(c * _CH, _CH)
        a_c = au_scr[rows, :]
        rel = lax.dot_general(a_c, embT, _STD,
                              preferred_element_type=jnp.float32)
        rel = jnp.where(valid, rel, -jnp.inf)
        tmax = jnp.max(rel, axis=1, keepdims=True)
        m_old = m_scr[rows, :]
        m_new = jnp.maximum(m_old, tmax)
        mask = (rel == m_new).astype(jnp.float32)
        contrib = lax.dot_general(mask, emb, _STD,
                                  preferred_element_type=jnp.float32)
        acc_scr[rows, :] = jnp.where(tmax > m_old,
                                     contrib, acc_scr[rows, :] + contrib)
        m_scr[rows, :] = m_new
        return carry

    lax.fori_loop(0, n_ch, _chunk, 0)

    @pl.when(t == _T - 1)
    def _fin():
        uio = lax.broadcasted_iota(jnp.int32, (1, _U), 1)
        selb = (rb_scr[...].astype(jnp.int32) == uio).astype(jnp.float32)
        out_ref[...] = lax.dot_general(selb, acc_scr[...], _STD,
                                       precision=_HI)


def kernel(items_indices, constant_table, adaptive_table):
    idx2 = items_indices.reshape(_B, 1)
    tpad = jnp.pad(constant_table, ((0, _V_PAD - _V), (0, 0)))
    tT = tpad.T
    return pl.pallas_call(
        _body,
        grid=(_T,),
        in_specs=[
            pl.BlockSpec((_B, 1), lambda t: (0, 0)),
            pl.BlockSpec((_A, _D), lambda t: (0, 0)),
            pl.BlockSpec((_D, _VT), lambda t: (0, t)),
            pl.BlockSpec((_VT, _D), lambda t: (t, 0)),
        ],
        out_specs=pl.BlockSpec((_B, _D), lambda t: (0, 0)),
        out_shape=jax.ShapeDtypeStruct((_B, _D), jnp.float32),
        scratch_shapes=[pltpu.VMEM((_U, _D), jnp.float32),
                        pltpu.VMEM((_U, 1), jnp.float32),
                        pltpu.VMEM((_U, _D), jnp.float32),
                        pltpu.VMEM((_B, 1), jnp.float32),
                        pltpu.SMEM((1,), jnp.int32)],
    )(idx2, adaptive_table, tT, tpad)


# speedup vs baseline: 5.7754x; 1.7763x over previous
import jax
import jax.numpy as jnp
from jax import lax
from jax.experimental import pallas as pl
from jax.experimental.pallas import tpu as pltpu

_V = 100000
_D = 16
_A = 1024
_B = 1024

_VT = 8192
_T = (_V + _VT - 1) // _VT
_V_PAD = _T * _VT
_CH = 32
_U = _A

_HI = lax.Precision.HIGHEST
_STD = (((1,), (0,)), ((), ()))
_RT = (((1,), (1,)), ((), ()))


def _body(idx_ref, adap_ref, emb_ref, out_ref,
          au_scr, m_scr, acc_scr, rb_scr, n_scr):
    t = pl.program_id(0)

    @pl.when(t == 0)
    def _init():
        eidx = jnp.maximum(idx_ref[...] - _V, 0)
        aiota = lax.broadcasted_iota(jnp.int32, (_B, _A), 1)
        oh_ba = (eidx == aiota).astype(jnp.float32)
        ones_r = jnp.ones((1, _B), jnp.float32)
        counts = lax.dot_general(ones_r, oh_ba, _STD, precision=_HI)
        pres = (counts > 0.0).astype(jnp.float32)
        i_col = lax.broadcasted_iota(jnp.int32, (_A, _A), 0)
        j_row = lax.broadcasted_iota(jnp.int32, (_A, _A), 1)
        gt = (i_col < j_row).astype(jnp.float32)
        lt = (j_row < i_col).astype(jnp.float32)
        rank_r = lax.dot_general(pres, gt, _STD, precision=_HI)
        rank_c = lax.dot_general(lt, pres, _RT, precision=_HI)
        riota = lax.broadcasted_iota(jnp.int32, (_U, _A), 0)
        sel = ((riota == rank_r.astype(jnp.int32)) &
               (pres > 0.0)).astype(jnp.float32)
        au_scr[...] = lax.dot_general(sel, adap_ref[...], _STD,
                                      precision=_HI)
        rb_scr[...] = lax.dot_general(oh_ba, rank_c, _STD,
                                      precision=_HI)
        n_scr[0] = jnp.sum(pres).astype(jnp.int32)
        m_scr[...] = jnp.full((_U, 1), -jnp.inf, jnp.float32)
        acc_scr[...] = jnp.zeros((_U, _D), jnp.float32)

    col = t * _VT + lax.broadcasted_iota(jnp.int32, (1, _VT), 1)
    valid = col < _V
    rowcol = t * _VT + lax.broadcasted_iota(jnp.int32, (_VT, 1), 0)
    emb = jnp.where(rowcol < _V, emb_ref[...], 0.0)
    n_ch = (n_scr[0] + _CH - 1) // _CH

    def _chunk(c, carry):
        rows = pl.ds(c * _CH, _CH)
        a_c = au_scr[rows, :]
        rel = lax.dot_general(a_c, emb, _RT,
                              preferred_element_type=jnp.float32)
        rel = jnp.where(valid, rel, -jnp.inf)
        tmax = jnp.max(rel, axis=1, keepdims=True)
        m_old = m_scr[rows, :]
        m_new = jnp.maximum(m_old, tmax)
        mask = (rel == m_new).astype(jnp.float32)
        contrib = lax.dot_general(mask, emb, _STD,
                                  preferred_element_type=jnp.float32)
        acc_scr[rows, :] = jnp.where(tmax > m_old,
                                     contrib, acc_scr[rows, :] + contrib)
        m_scr[rows, :] = m_new
        return carry

    lax.fori_loop(0, n_ch, _chunk, 0)

    @pl.when(t == _T - 1)
    def _fin():
        uio = lax.broadcasted_iota(jnp.int32, (1, _U), 1)
        selb = (rb_scr[...].astype(jnp.int32) == uio).astype(jnp.float32)
        out_ref[...] = lax.dot_general(selb, acc_scr[...], _STD,
                                       precision=_HI)


def kernel(items_indices, constant_table, adaptive_table):
    idx2 = items_indices.reshape(_B, 1)
    return pl.pallas_call(
        _body,
        grid=(_T,),
        in_specs=[
            pl.BlockSpec((_B, 1), lambda t: (0, 0)),
            pl.BlockSpec((_A, _D), lambda t: (0, 0)),
            pl.BlockSpec((_VT, _D), lambda t: (t, 0)),
        ],
        out_specs=pl.BlockSpec((_B, _D), lambda t: (0, 0)),
        out_shape=jax.ShapeDtypeStruct((_B, _D), jnp.float32),
        scratch_shapes=[pltpu.VMEM((_U, _D), jnp.float32),
                        pltpu.VMEM((_U, 1), jnp.float32),
                        pltpu.VMEM((_U, _D), jnp.float32),
                        pltpu.VMEM((_B, 1), jnp.float32),
                        pltpu.SMEM((1,), jnp.int32)],
    )(idx2, adaptive_table, constant_table)


# speedup vs baseline: 5.9989x; 1.0387x over previous
import jax
import jax.numpy as jnp
from jax import lax
from jax.experimental import pallas as pl
from jax.experimental.pallas import tpu as pltpu

_V = 100000
_D = 16
_A = 1024
_B = 1024

_VT = 8192
_T = (_V + _VT - 1) // _VT
_V_PAD = _T * _VT
_CH = 16
_U = _A

_HI = lax.Precision.HIGHEST
_STD = (((1,), (0,)), ((), ()))
_RT = (((1,), (1,)), ((), ()))


def _body(idx_ref, adap_ref, emb_ref, out_ref,
          au_scr, m_scr, acc_scr, rb_scr, n_scr):
    t = pl.program_id(0)

    @pl.when(t == 0)
    def _init():
        eidx = jnp.maximum(idx_ref[...] - _V, 0)
        aiota = lax.broadcasted_iota(jnp.int32, (_B, _A), 1)
        oh_ba = (eidx == aiota).astype(jnp.float32)
        ones_r = jnp.ones((1, _B), jnp.float32)
        counts = lax.dot_general(ones_r, oh_ba, _STD, precision=_HI)
        pres = (counts > 0.0).astype(jnp.float32)
        i_col = lax.broadcasted_iota(jnp.int32, (_A, _A), 0)
        j_row = lax.broadcasted_iota(jnp.int32, (_A, _A), 1)
        gt = (i_col < j_row).astype(jnp.float32)
        lt = (j_row < i_col).astype(jnp.float32)
        rank_r = lax.dot_general(pres, gt, _STD, precision=_HI)
        rank_c = lax.dot_general(lt, pres, _RT, precision=_HI)
        riota = lax.broadcasted_iota(jnp.int32, (_U, _A), 0)
        sel = ((riota == rank_r.astype(jnp.int32)) &
               (pres > 0.0)).astype(jnp.float32)
        au_scr[...] = lax.dot_general(sel, adap_ref[...], _STD,
                                      precision=_HI)
        rb_scr[...] = lax.dot_general(oh_ba, rank_c, _STD,
                                      precision=_HI)
        n_scr[0] = jnp.sum(pres).astype(jnp.int32)
        m_scr[...] = jnp.full((_U, 1), -jnp.inf, jnp.float32)
        acc_scr[...] = jnp.zeros((_U, _D), jnp.float32)

    is_last = t == _T - 1
    emb = lax.cond(
        is_last,
        lambda: jnp.where(
            lax.broadcasted_iota(jnp.int32, (_VT, 1), 0) < _V - (_T - 1) * _VT,
            emb_ref[...], 0.0),
        lambda: emb_ref[...])
    n_ch = (n_scr[0] + _CH - 1) // _CH

    def _chunk(c, carry):
        rows = pl.ds(c * _CH, _CH)
        a_c = au_scr[rows, :]
        rel = lax.dot_general(a_c, emb, _RT,
                              preferred_element_type=jnp.float32)
        rel = lax.cond(
            is_last,
            lambda r: jnp.where(
                lax.broadcasted_iota(jnp.int32, (1, _VT), 1) < _V - (_T - 1) * _VT,
                r, -jnp.inf),
            lambda r: r, rel)
        tmax = jnp.max(rel, axis=1, keepdims=True)
        m_old = m_scr[rows, :]
        m_new = jnp.maximum(m_old, tmax)
        mask = (rel == m_new).astype(jnp.float32)
        contrib = lax.dot_general(mask, emb, _STD,
                                  preferred_element_type=jnp.float32)
        acc_scr[rows, :] = jnp.where(tmax > m_old,
                                     contrib, acc_scr[rows, :] + contrib)
        m_scr[rows, :] = m_new
        return carry

    lax.fori_loop(0, n_ch, _chunk, 0)

    @pl.when(t == _T - 1)
    def _fin():
        uio = lax.broadcasted_iota(jnp.int32, (1, _U), 1)
        selb = (rb_scr[...].astype(jnp.int32) == uio).astype(jnp.float32)
        out_ref[...] = lax.dot_general(selb, acc_scr[...], _STD,
                                       precision=_HI)


def kernel(items_indices, constant_table, adaptive_table):
    idx2 = items_indices.reshape(_B, 1)
    return pl.pallas_call(
        _body,
        grid=(_T,),
        in_specs=[
            pl.BlockSpec((_B, 1), lambda t: (0, 0)),
            pl.BlockSpec((_A, _D), lambda t: (0, 0)),
            pl.BlockSpec((_VT, _D), lambda t: (t, 0)),
        ],
        out_specs=pl.BlockSpec((_B, _D), lambda t: (0, 0)),
        out_shape=jax.ShapeDtypeStruct((_B, _D), jnp.float32),
        scratch_shapes=[pltpu.VMEM((_U, _D), jnp.float32),
                        pltpu.VMEM((_U, 1), jnp.float32),
                        pltpu.VMEM((_U, _D), jnp.float32),
                        pltpu.VMEM((_B, 1), jnp.float32),
                        pltpu.SMEM((1,), jnp.int32)],
    )(idx2, adaptive_table, constant_table)


# speedup vs baseline: 6.9403x; 1.1569x over previous
import jax
import jax.numpy as jnp
from jax import lax
from jax.experimental import pallas as pl
from jax.experimental.pallas import tpu as pltpu

_V = 100000
_D = 16
_A = 1024
_B = 1024

_VT = 8192
_T = (_V + _VT - 1) // _VT
_V_PAD = _T * _VT
_CH = 16
_U = _A

_HI = lax.Precision.DEFAULT
_STD = (((1,), (0,)), ((), ()))
_RT = (((1,), (1,)), ((), ()))


def _body(idx_ref, adap_ref, emb_ref, out_ref,
          au_scr, m_scr, acc_scr, rb_scr, n_scr):
    t = pl.program_id(0)

    @pl.when(t == 0)
    def _init():
        eidx = jnp.maximum(idx_ref[...] - _V, 0)
        aiota = lax.broadcasted_iota(jnp.int32, (_B, _A), 1)
        oh_ba = (eidx == aiota).astype(jnp.float32)
        ones_r = jnp.ones((1, _B), jnp.float32)
        counts = lax.dot_general(ones_r, oh_ba, _STD, precision=_HI)
        pres = (counts > 0.0).astype(jnp.float32)
        i_col = lax.broadcasted_iota(jnp.int32, (_A, _A), 0)
        j_row = lax.broadcasted_iota(jnp.int32, (_A, _A), 1)
        gt = (i_col < j_row).astype(jnp.float32)
        lt = (j_row < i_col).astype(jnp.float32)
        rank_r = lax.dot_general(pres, gt, _STD, precision=_HI)
        rank_c = lax.dot_general(lt, pres, _RT, precision=_HI)
        riota = lax.broadcasted_iota(jnp.int32, (_U, _A), 0)
        sel = ((riota == rank_r.astype(jnp.int32)) &
               (pres > 0.0)).astype(jnp.float32)
        au_scr[...] = lax.dot_general(sel, adap_ref[...], _STD,
                                      precision=_HI)
        rb_scr[...] = lax.dot_general(oh_ba, rank_c, _STD,
                                      precision=_HI)
        n_scr[0] = jnp.sum(pres).astype(jnp.int32)
        m_scr[...] = jnp.full((_U, 1), -jnp.inf, jnp.float32)
        acc_scr[...] = jnp.zeros((_U, _D), jnp.float32)

    is_last = t == _T - 1
    emb = lax.cond(
        is_last,
        lambda: jnp.where(
            lax.broadcasted_iota(jnp.int32, (_VT, 1), 0) < _V - (_T - 1) * _VT,
            emb_ref[...], 0.0),
        lambda: emb_ref[...])
    n_ch = (n_scr[0] + _CH - 1) // _CH

    def _chunk(c, carry):
        rows = pl.ds(c * _CH, _CH)
        a_c = au_scr[rows, :]
        rel = lax.dot_general(a_c, emb, _RT,
                              preferred_element_type=jnp.float32)
        rel = lax.cond(
            is_last,
            lambda r: jnp.where(
                lax.broadcasted_iota(jnp.int32, (1, _VT), 1) < _V - (_T - 1) * _VT,
                r, -jnp.inf),
            lambda r: r, rel)
        tmax = jnp.max(rel, axis=1, keepdims=True)
        m_old = m_scr[rows, :]
        m_new = jnp.maximum(m_old, tmax)
        mask = (rel == m_new).astype(jnp.float32)
        contrib = lax.dot_general(mask, emb, _STD,
                                  preferred_element_type=jnp.float32)
        acc_scr[rows, :] = jnp.where(tmax > m_old,
                                     contrib, acc_scr[rows, :] + contrib)
        m_scr[rows, :] = m_new
        return carry

    lax.fori_loop(0, n_ch, _chunk, 0)

    @pl.when(t == _T - 1)
    def _fin():
        uio = lax.broadcasted_iota(jnp.int32, (1, _U), 1)
        selb = (rb_scr[...].astype(jnp.int32) == uio).astype(jnp.float32)
        out_ref[...] = lax.dot_general(selb, acc_scr[...], _STD,
                                       precision=_HI)


def kernel(items_indices, constant_table, adaptive_table):
    idx2 = items_indices.reshape(_B, 1)
    return pl.pallas_call(
        _body,
        grid=(_T,),
        in_specs=[
            pl.BlockSpec((_B, 1), lambda t: (0, 0)),
            pl.BlockSpec((_A, _D), lambda t: (0, 0)),
            pl.BlockSpec((_VT, _D), lambda t: (t, 0)),
        ],
        out_specs=pl.BlockSpec((_B, _D), lambda t: (0, 0)),
        out_shape=jax.ShapeDtypeStruct((_B, _D), jnp.float32),
        scratch_shapes=[pltpu.VMEM((_U, _D), jnp.float32),
                        pltpu.VMEM((_U, 1), jnp.float32),
                        pltpu.VMEM((_U, _D), jnp.float32),
                        pltpu.VMEM((_B, 1), jnp.float32),
                        pltpu.SMEM((1,), jnp.int32)],
    )(idx2, adaptive_table, constant_table)


# speedup vs baseline: 6.9406x; 1.0000x over previous
import jax
import jax.numpy as jnp
from jax import lax
from jax.experimental import pallas as pl
from jax.experimental.pallas import tpu as pltpu

_V = 100000
_D = 16
_A = 1024
_B = 1024

_VT = 16384
_T = (_V + _VT - 1) // _VT
_V_PAD = _T * _VT
_CH = 16
_U = _A

_HI = lax.Precision.DEFAULT
_STD = (((1,), (0,)), ((), ()))
_RT = (((1,), (1,)), ((), ()))


def _body(idx_ref, adap_ref, emb_ref, out_ref,
          au_scr, m_scr, acc_scr, rb_scr, n_scr):
    t = pl.program_id(0)

    @pl.when(t == 0)
    def _init():
        eidx = jnp.maximum(idx_ref[...] - _V, 0)
        aiota = lax.broadcasted_iota(jnp.int32, (_B, _A), 1)
        oh_ba = (eidx == aiota).astype(jnp.float32)
        ones_r = jnp.ones((1, _B), jnp.float32)
        counts = lax.dot_general(ones_r, oh_ba, _STD, precision=_HI)
        pres = (counts > 0.0).astype(jnp.float32)
        i_col = lax.broadcasted_iota(jnp.int32, (_A, _A), 0)
        j_row = lax.broadcasted_iota(jnp.int32, (_A, _A), 1)
        gt = (i_col < j_row).astype(jnp.float32)
        lt = (j_row < i_col).astype(jnp.float32)
        rank_r = lax.dot_general(pres, gt, _STD, precision=_HI)
        rank_c = lax.dot_general(lt, pres, _RT, precision=_HI)
        riota = lax.broadcasted_iota(jnp.int32, (_U, _A), 0)
        sel = ((riota == rank_r.astype(jnp.int32)) &
               (pres > 0.0)).astype(jnp.float32)
        au_scr[...] = lax.dot_general(sel, adap_ref[...], _STD,
                                      precision=_HI)
        rb_scr[...] = lax.dot_general(oh_ba, rank_c, _STD,
                                      precision=_HI)
        n_scr[0] = jnp.sum(pres).astype(jnp.int32)
        m_scr[...] = jnp.full((_U, 1), -jnp.inf, jnp.float32)
        acc_scr[...] = jnp.zeros((_U, _D), jnp.float32)

    is_last = t == _T - 1
    emb = lax.cond(
        is_last,
        lambda: jnp.where(
            lax.broadcasted_iota(jnp.int32, (_VT, 1), 0) < _V - (_T - 1) * _VT,
            emb_ref[...], 0.0),
        lambda: emb_ref[...])
    n_ch = (n_scr[0] + _CH - 1) // _CH

    def _chunk(c, carry):
        rows = pl.ds(c * _CH, _CH)
        a_c = au_scr[rows, :]
        rel = lax.dot_general(a_c, emb, _RT,
                              preferred_element_type=jnp.float32)
        rel = lax.cond(
            is_last,
            lambda r: jnp.where(
                lax.broadcasted_iota(jnp.int32, (1, _VT), 1) < _V - (_T - 1) * _VT,
                r, -jnp.inf),
            lambda r: r, rel)
        tmax = jnp.max(rel, axis=1, keepdims=True)
        m_old = m_scr[rows, :]
        m_new = jnp.maximum(m_old, tmax)
        mask = (rel == m_new).astype(jnp.float32)
        contrib = lax.dot_general(mask, emb, _STD,
                                  preferred_element_type=jnp.float32)
        acc_scr[rows, :] = jnp.where(tmax > m_old,
                                     contrib, acc_scr[rows, :] + contrib)
        m_scr[rows, :] = m_new
        return carry

    lax.fori_loop(0, n_ch, _chunk, 0)

    @pl.when(t == _T - 1)
    def _fin():
        uio = lax.broadcasted_iota(jnp.int32, (1, _U), 1)
        selb = (rb_scr[...].astype(jnp.int32) == uio).astype(jnp.float32)
        out_ref[...] = lax.dot_general(selb, acc_scr[...], _STD,
                                       precision=_HI)


def kernel(items_indices, constant_table, adaptive_table):
    idx2 = items_indices.reshape(_B, 1)
    return pl.pallas_call(
        _body,
        grid=(_T,),
        in_specs=[
            pl.BlockSpec((_B, 1), lambda t: (0, 0)),
            pl.BlockSpec((_A, _D), lambda t: (0, 0)),
            pl.BlockSpec((_VT, _D), lambda t: (t, 0)),
        ],
        out_specs=pl.BlockSpec((_B, _D), lambda t: (0, 0)),
        out_shape=jax.ShapeDtypeStruct((_B, _D), jnp.float32),
        scratch_shapes=[pltpu.VMEM((_U, _D), jnp.float32),
                        pltpu.VMEM((_U, 1), jnp.float32),
                        pltpu.VMEM((_U, _D), jnp.float32),
                        pltpu.VMEM((_B, 1), jnp.float32),
                        pltpu.SMEM((1,), jnp.int32)],
    )(idx2, adaptive_table, constant_table)
